# baseline (speedup 1.0000x reference)
import functools

import numpy as np
import jax
import jax.numpy as jnp
from jax.experimental import pallas as pl
from jax.experimental.pallas import tpu as pltpu


def _rdft_ac_mats_bf16(seq_len):
    import ml_dtypes
    n = np.arange(seq_len)
    k = np.arange(1, seq_len // 2 + 1)
    ang = 2.0 * np.pi * np.outer(n, k) / seq_len
    s = 1.0 / np.sqrt(seq_len)
    cos = (np.cos(ang) * s).astype(np.float32).astype(ml_dtypes.bfloat16)
    sin = (-np.sin(ang) * s).astype(np.float32).astype(ml_dtypes.bfloat16)
    return cos, sin


def _seg_ones(lt, f):
    import ml_dtypes
    s = np.zeros((lt * f, lt), np.float32)
    for m in range(lt):
        s[m * f:(m + 1) * f, m] = 1.0
    return s.astype(ml_dtypes.bfloat16)


def _stage1_kernel(x_ref, wrep_ref, bs_ref, s01_ref, cos_ref, sin_ref,
                   xr_ref, xi_ref, accr, acci, *, n_steps):
    step = pl.program_id(0)

    @pl.when(step == 0)
    def _init():
        accr[...] = jnp.zeros_like(accr)
        acci[...] = jnp.zeros_like(acci)

    y = x_ref[...] * wrep_ref[...]
    y_hi = y.astype(jnp.bfloat16)
    y_lo = (y - y_hi.astype(jnp.float32)).astype(jnp.bfloat16)
    s01 = s01_ref[...]
    h = (jnp.dot(y_hi, s01, preferred_element_type=jnp.float32)
         + jnp.dot(y_lo, s01, preferred_element_type=jnp.float32)
         + bs_ref[0, 0])
    accr[...] += jnp.dot(h, cos_ref[...], preferred_element_type=jnp.float32)
    acci[...] += jnp.dot(h, sin_ref[...], preferred_element_type=jnp.float32)

    @pl.when(step == n_steps - 1)
    def _flush():
        xr_ref[...] = accr[...].astype(jnp.bfloat16)
        xi_ref[...] = acci[...].astype(jnp.bfloat16)


def _stage2_kernel(xr_ref, xi_ref, w1_ref, b1_ref, w2_ref, b2_ref, wg_ref,
                   out_ref, accr, acci, *, n_steps, n_patch):
    step = pl.program_id(0)

    @pl.when(step == 0)
    def _init():
        accr[...] = jnp.zeros_like(accr)
        acci[...] = jnp.zeros_like(acci)

    xr = xr_ref[...]
    xi = xi_ref[...]
    w1r = w1_ref[0]
    w1i = w1_ref[1]
    o1r = jax.nn.relu(jnp.dot(xr, w1r, preferred_element_type=jnp.float32)
                      - jnp.dot(xi, w1i, preferred_element_type=jnp.float32)
                      + b1_ref[0]).astype(jnp.bfloat16)
    o1i = jax.nn.relu(jnp.dot(xi, w1r, preferred_element_type=jnp.float32)
                      + jnp.dot(xr, w1i, preferred_element_type=jnp.float32)
                      + b1_ref[1]).astype(jnp.bfloat16)
    w2r = w2_ref[0]
    w2i = w2_ref[1]
    accr[...] += (jnp.dot(o1r, w2r, preferred_element_type=jnp.float32)
                  - jnp.dot(o1i, w2i, preferred_element_type=jnp.float32))
    acci[...] += (jnp.dot(o1i, w2r, preferred_element_type=jnp.float32)
                  + jnp.dot(o1r, w2i, preferred_element_type=jnp.float32))

    @pl.when(step == n_steps - 1)
    def _epilogue():
        o2r = accr[...] + b2_ref[0]
        o2i = acci[...] + b2_ref[1]
        amp = jnp.sqrt(o2r * o2r + o2i * o2i).astype(jnp.bfloat16)
        logits = jnp.dot(amp, wg_ref[...], preferred_element_type=jnp.float32)
        b = logits.shape[0]
        col = jax.lax.broadcasted_iota(jnp.int32, (b, n_patch), 1)
        m1 = jnp.max(logits, axis=-1, keepdims=True)
        i1 = jnp.min(jnp.where(logits == m1, col, n_patch), axis=-1, keepdims=True)
        mask1 = col == i1
        rest = jnp.where(mask1, -jnp.inf, logits)
        m2 = jnp.max(rest, axis=-1, keepdims=True)
        i2 = jnp.min(jnp.where(rest == m2, col, n_patch), axis=-1, keepdims=True)
        mask2 = col == i2
        p1 = jax.nn.sigmoid(m1 - m2)
        p2 = jax.nn.sigmoid(m2 - m1)
        zero = jnp.zeros_like(logits)
        out_ref[...] = jnp.where(mask1, p1, jnp.where(mask2, p2, zero))


def kernel(x, training, W_start, b_start, w1, b1, w2, b2, w_gate):
    del training
    B, L, F = x.shape
    NF = w1.shape[1]
    H = w1.shape[2]
    P = w_gate.shape[1]

    cos_bf, sin_bf = _rdft_ac_mats_bf16(L)

    LT = 128
    n1 = L // LT
    x2 = x.reshape(B, L * F)
    wv = W_start.reshape(F).astype(jnp.float32)
    wrep = jnp.tile(wv, LT).reshape(1, LT * F)
    bs = b_start.reshape(1, 1).astype(jnp.float32)
    s01 = jnp.asarray(_seg_ones(LT, F))

    xr, xi = pl.pallas_call(
        functools.partial(_stage1_kernel, n_steps=n1),
        grid=(n1,),
        in_specs=[
            pl.BlockSpec((B, LT * F), lambda l: (0, l)),
            pl.BlockSpec((1, LT * F), lambda l: (0, 0)),
            pl.BlockSpec((1, 1), lambda l: (0, 0)),
            pl.BlockSpec((LT * F, LT), lambda l: (0, 0)),
            pl.BlockSpec((LT, NF), lambda l: (l, 0)),
            pl.BlockSpec((LT, NF), lambda l: (l, 0)),
        ],
        out_specs=[
            pl.BlockSpec((B, NF), lambda l: (0, 0)),
            pl.BlockSpec((B, NF), lambda l: (0, 0)),
        ],
        out_shape=[
            jax.ShapeDtypeStruct((B, NF), jnp.bfloat16),
            jax.ShapeDtypeStruct((B, NF), jnp.bfloat16),
        ],
        scratch_shapes=[
            pltpu.VMEM((B, NF), jnp.float32),
            pltpu.VMEM((B, NF), jnp.float32),
        ],
    )(x2, wrep, bs, s01, jnp.asarray(cos_bf), jnp.asarray(sin_bf))

    HT = 512
    n2 = H // HT
    gates = pl.pallas_call(
        functools.partial(_stage2_kernel, n_steps=n2, n_patch=P),
        grid=(n2,),
        in_specs=[
            pl.BlockSpec((B, NF), lambda hh: (0, 0)),
            pl.BlockSpec((B, NF), lambda hh: (0, 0)),
            pl.BlockSpec((2, NF, HT), lambda hh: (0, 0, hh)),
            pl.BlockSpec((2, HT), lambda hh: (0, hh)),
            pl.BlockSpec((2, HT, NF), lambda hh: (0, hh, 0)),
            pl.BlockSpec((2, NF), lambda hh: (0, 0)),
            pl.BlockSpec((NF, P), lambda hh: (0, 0)),
        ],
        out_specs=pl.BlockSpec((B, P), lambda hh: (0, 0)),
        out_shape=jax.ShapeDtypeStruct((B, P), jnp.float32),
        scratch_shapes=[
            pltpu.VMEM((B, NF), jnp.float32),
            pltpu.VMEM((B, NF), jnp.float32),
        ],
    )(xr, xi, w1, b1, w2, b2, w_gate)

    return gates


# speedup vs baseline: 1.1084x; 1.1084x over previous
import functools

import numpy as np
import jax
import jax.numpy as jnp
from jax.experimental import pallas as pl
from jax.experimental.pallas import tpu as pltpu


def _rdft_ac_mats_bf16(seq_len):
    import ml_dtypes
    n = np.arange(seq_len)
    k = np.arange(1, seq_len // 2 + 1)
    ang = 2.0 * np.pi * np.outer(n, k) / seq_len
    s = 1.0 / np.sqrt(seq_len)
    cos = (np.cos(ang) * s).astype(np.float32).astype(ml_dtypes.bfloat16)
    sin = (-np.sin(ang) * s).astype(np.float32).astype(ml_dtypes.bfloat16)
    return cos, sin


def _stage1_kernel(x_ref, ws_ref, bs_ref, cos_ref, sin_ref,
                   xr_ref, xi_ref, accr, acci, *, n_steps):
    step = pl.program_id(0)

    @pl.when(step == 0)
    def _init():
        accr[...] = jnp.zeros_like(accr)
        acci[...] = jnp.zeros_like(acci)

    xt = x_ref[...]
    w = ws_ref[...]
    h = jnp.sum(xt * w, axis=-1) + bs_ref[0, 0]
    accr[...] += jnp.dot(h, cos_ref[...], preferred_element_type=jnp.float32)
    acci[...] += jnp.dot(h, sin_ref[...], preferred_element_type=jnp.float32)

    @pl.when(step == n_steps - 1)
    def _flush():
        xr_ref[...] = accr[...].astype(jnp.bfloat16)
        xi_ref[...] = acci[...].astype(jnp.bfloat16)


def _stage2_kernel(xr_ref, xi_ref, w1_ref, b1_ref, w2_ref, b2_ref, wg_ref,
                   out_ref, accr, acci, *, n_steps, n_patch):
    step = pl.program_id(0)

    @pl.when(step == 0)
    def _init():
        accr[...] = jnp.zeros_like(accr)
        acci[...] = jnp.zeros_like(acci)

    xr = xr_ref[...]
    xi = xi_ref[...]
    w1r = w1_ref[0]
    w1i = w1_ref[1]
    o1r = jax.nn.relu(jnp.dot(xr, w1r, preferred_element_type=jnp.float32)
                      - jnp.dot(xi, w1i, preferred_element_type=jnp.float32)
                      + b1_ref[0]).astype(jnp.bfloat16)
    o1i = jax.nn.relu(jnp.dot(xi, w1r, preferred_element_type=jnp.float32)
                      + jnp.dot(xr, w1i, preferred_element_type=jnp.float32)
                      + b1_ref[1]).astype(jnp.bfloat16)
    w2r = w2_ref[0]
    w2i = w2_ref[1]
    accr[...] += (jnp.dot(o1r, w2r, preferred_element_type=jnp.float32)
                  - jnp.dot(o1i, w2i, preferred_element_type=jnp.float32))
    acci[...] += (jnp.dot(o1i, w2r, preferred_element_type=jnp.float32)
                  + jnp.dot(o1r, w2i, preferred_element_type=jnp.float32))

    @pl.when(step == n_steps - 1)
    def _epilogue():
        o2r = accr[...] + b2_ref[0]
        o2i = acci[...] + b2_ref[1]
        amp = jnp.sqrt(o2r * o2r + o2i * o2i).astype(jnp.bfloat16)
        logits = jnp.dot(amp, wg_ref[...], preferred_element_type=jnp.float32)
        b = logits.shape[0]
        col = jax.lax.broadcasted_iota(jnp.int32, (b, n_patch), 1)
        m1 = jnp.max(logits, axis=-1, keepdims=True)
        i1 = jnp.min(jnp.where(logits == m1, col, n_patch), axis=-1, keepdims=True)
        mask1 = col == i1
        rest = jnp.where(mask1, -jnp.inf, logits)
        m2 = jnp.max(rest, axis=-1, keepdims=True)
        i2 = jnp.min(jnp.where(rest == m2, col, n_patch), axis=-1, keepdims=True)
        mask2 = col == i2
        p1 = jax.nn.sigmoid(m1 - m2)
        p2 = jax.nn.sigmoid(m2 - m1)
        zero = jnp.zeros_like(logits)
        out_ref[...] = jnp.where(mask1, p1, jnp.where(mask2, p2, zero))


def kernel(x, training, W_start, b_start, w1, b1, w2, b2, w_gate):
    del training
    B, L, F = x.shape
    NF = w1.shape[1]
    H = w1.shape[2]
    P = w_gate.shape[1]

    cos_bf, sin_bf = _rdft_ac_mats_bf16(L)

    LT = 256
    n1 = L // LT
    ws = W_start.reshape(1, 1, F).astype(jnp.float32)
    bs = b_start.reshape(1, 1).astype(jnp.float32)

    xr, xi = pl.pallas_call(
        functools.partial(_stage1_kernel, n_steps=n1),
        grid=(n1,),
        in_specs=[
            pl.BlockSpec((B, LT, F), lambda l: (0, l, 0)),
            pl.BlockSpec((1, 1, F), lambda l: (0, 0, 0)),
            pl.BlockSpec((1, 1), lambda l: (0, 0)),
            pl.BlockSpec((LT, NF), lambda l: (l, 0)),
            pl.BlockSpec((LT, NF), lambda l: (l, 0)),
        ],
        out_specs=[
            pl.BlockSpec((B, NF), lambda l: (0, 0)),
            pl.BlockSpec((B, NF), lambda l: (0, 0)),
        ],
        out_shape=[
            jax.ShapeDtypeStruct((B, NF), jnp.bfloat16),
            jax.ShapeDtypeStruct((B, NF), jnp.bfloat16),
        ],
        scratch_shapes=[
            pltpu.VMEM((B, NF), jnp.float32),
            pltpu.VMEM((B, NF), jnp.float32),
        ],
    )(x, ws, bs, jnp.asarray(cos_bf), jnp.asarray(sin_bf))

    HT = 512
    n2 = H // HT
    gates = pl.pallas_call(
        functools.partial(_stage2_kernel, n_steps=n2, n_patch=P),
        grid=(n2,),
        in_specs=[
            pl.BlockSpec((B, NF), lambda hh: (0, 0)),
            pl.BlockSpec((B, NF), lambda hh: (0, 0)),
            pl.BlockSpec((2, NF, HT), lambda hh: (0, 0, hh)),
            pl.BlockSpec((2, HT), lambda hh: (0, hh)),
            pl.BlockSpec((2, HT, NF), lambda hh: (0, hh, 0)),
            pl.BlockSpec((2, NF), lambda hh: (0, 0)),
            pl.BlockSpec((NF, P), lambda hh: (0, 0)),
        ],
        out_specs=pl.BlockSpec((B, P), lambda hh: (0, 0)),
        out_shape=jax.ShapeDtypeStruct((B, P), jnp.float32),
        scratch_shapes=[
            pltpu.VMEM((B, NF), jnp.float32),
            pltpu.VMEM((B, NF), jnp.float32),
        ],
    )(xr, xi, w1, b1, w2, b2, w_gate)

    return gates
